# Initial kernel scaffold; baseline (speedup 1.0000x reference)
import jax
import jax.numpy as jnp
from jax.experimental import pallas as pl


def kernel(token_ids, weights):
    raise NotImplementedError("write your pallas kernel here")


# speedup vs baseline: 1.1034x; 1.1034x over previous
import functools

import jax
import jax.numpy as jnp
from jax import lax
from jax.experimental import pallas as pl
from jax.experimental.pallas import tpu as pltpu
from jax.experimental.pallas import tpu_sc as plsc

EMBED_D = 32
NUM_CORES = 2
NUM_SUBCORES = 16
NW = NUM_CORES * NUM_SUBCORES
G = 128
CH = 10


@functools.cache
def _build(B: int):
    b_per_w = B // NW
    n_g = b_per_w // G
    n_chunks = n_g // CH
    C = G * CH

    mesh = plsc.VectorSubcoreMesh(core_axis_name="c", subcore_axis_name="s")

    @functools.partial(
        pl.kernel,
        mesh=mesh,
        compiler_params=pltpu.CompilerParams(use_tc_tiling_on_sc=False),
        out_type=jax.ShapeDtypeStruct((B, EMBED_D), jnp.float32),
        scratch_types=[
            pltpu.VMEM((n_g, G), jnp.int32),
            pltpu.VMEM((C, EMBED_D), jnp.float32),
            pltpu.SemaphoreType.DMA,
        ],
    )
    def emb_kernel(idx_hbm, table_hbm, out_hbm, idx_v, rows_v, sem):
        wid = lax.axis_index("s") * NUM_CORES + lax.axis_index("c")
        pltpu.sync_copy(idx_hbm.at[pl.ds(wid * n_g, n_g)], idx_v)
        base = wid * b_per_w

        def chunk_body(c, carry):
            copies = []
            for j in range(CH):
                copies.append(
                    pltpu.async_copy(
                        table_hbm.at[idx_v.at[c * CH + j]],
                        rows_v.at[pl.ds(j * G, G)],
                        sem,
                    )
                )
            for cp in copies:
                cp.wait()
            pltpu.sync_copy(rows_v, out_hbm.at[pl.ds(base + c * C, C)])
            return carry

        lax.fori_loop(0, n_chunks, chunk_body, 0)

    return emb_kernel


def kernel(token_ids, weights):
    B = token_ids.shape[0] * token_ids.shape[1]
    idx = token_ids.reshape(NW * (B // NW // G), G).astype(jnp.int32)
    out = _build(B)(idx, weights)
    return out.reshape(token_ids.shape + (EMBED_D,))


# speedup vs baseline: 1.1118x; 1.0076x over previous
import functools

import jax
import jax.numpy as jnp
from jax import lax
from jax.experimental import pallas as pl
from jax.experimental.pallas import tpu as pltpu
from jax.experimental.pallas import tpu_sc as plsc

EMBED_D = 32
NUM_CORES = 2
NUM_SUBCORES = 16
NW = NUM_CORES * NUM_SUBCORES
G = 128
CH = 10


@functools.cache
def _build(B: int):
    b_per_w = B // NW
    n_g = b_per_w // G
    C = G * CH
    n_chunks = n_g // CH
    nh = n_chunks // 2

    mesh = plsc.VectorSubcoreMesh(core_axis_name="c", subcore_axis_name="s")

    @functools.partial(
        pl.kernel,
        mesh=mesh,
        compiler_params=pltpu.CompilerParams(use_tc_tiling_on_sc=False),
        out_type=jax.ShapeDtypeStruct((B, EMBED_D), jnp.float32),
        scratch_types=[
            pltpu.VMEM((n_g, G), jnp.int32),
            pltpu.VMEM((C, EMBED_D), jnp.float32),
            pltpu.VMEM((C, EMBED_D), jnp.float32),
            pltpu.SemaphoreType.DMA,
            pltpu.SemaphoreType.DMA,
            pltpu.SemaphoreType.DMA,
            pltpu.SemaphoreType.DMA,
        ],
    )
    def emb_kernel(idx_hbm, table_hbm, out_hbm, idx_v, rows0, rows1,
                   sem_g0, sem_g1, sem_s0, sem_s1):
        wid = lax.axis_index("s") * NUM_CORES + lax.axis_index("c")
        pltpu.sync_copy(idx_hbm.at[pl.ds(wid * n_g, n_g)], idx_v)
        base = wid * b_per_w

        def fire(c, rows, sem):
            for j in range(CH):
                pltpu.async_copy(
                    table_hbm.at[idx_v.at[c * CH + j]],
                    rows.at[pl.ds(j * G, G)],
                    sem,
                )

        def drain_gather(rows, sem):
            pltpu.make_async_copy(table_hbm.at[pl.ds(0, C)], rows, sem).wait()

        def start_store(c, rows, sem):
            pltpu.async_copy(rows, out_hbm.at[pl.ds(base + c * C, C)], sem)

        def drain_store(rows, sem):
            pltpu.make_async_copy(rows, out_hbm.at[pl.ds(0, C)], sem).wait()

        fire(0, rows0, sem_g0)

        def body(i, carry):
            c0 = 2 * i
            c1 = c0 + 1

            @pl.when(i > 0)
            def _():
                drain_store(rows1, sem_s1)
            fire(c1, rows1, sem_g1)
            drain_gather(rows0, sem_g0)
            start_store(c0, rows0, sem_s0)
            drain_store(rows0, sem_s0)

            @pl.when(i + 1 < nh)
            def _():
                fire(c0 + 2, rows0, sem_g0)
            drain_gather(rows1, sem_g1)
            start_store(c1, rows1, sem_s1)
            return carry

        lax.fori_loop(0, nh, body, 0)
        drain_store(rows1, sem_s1)

    return emb_kernel


def kernel(token_ids, weights):
    B = token_ids.shape[0] * token_ids.shape[1]
    idx = token_ids.reshape(NW * (B // NW // G), G).astype(jnp.int32)
    out = _build(B)(idx, weights)
    return out.reshape(token_ids.shape + (EMBED_D,))
